# Initial kernel scaffold; baseline (speedup 1.0000x reference)
import jax
import jax.numpy as jnp
from jax.experimental import pallas as pl


def kernel(x, edge_index, Wl1, bl1, Wr1, Wl2, bl2, Wr2, ln_g, ln_b, Wi, bi, Wh, bh, tW1, tb1, tW2, tb2, pW1, pb1, pW2, pb2):
    raise NotImplementedError("write your pallas kernel here")


# speedup vs baseline: 7.4842x; 7.4842x over previous
import functools

import jax
import jax.numpy as jnp
from jax import lax
from jax.experimental import pallas as pl
from jax.experimental.pallas import tpu as pltpu
from jax.experimental.pallas import tpu_sc as plsc

T, N, E, F, H = 20, 10000, 320000, 128, 128
NC, NS, NW = 2, 16, 32
K = 80
C = (E // NS) // K
NB = 2
NPAD = 10240
ND = NPAD // NC
AP = ND + 128
ZS = AP // NS
DS = ND // NS
DW = 16
CD = (E // NW) // K
R = 2048
G = NPAD // R
EB = E // 128


def _sc_mesh():
    return plsc.VectorSubcoreMesh(
        core_axis_name="c", subcore_axis_name="s", num_cores=NC, num_subcores=NS
    )


@functools.partial(
    pl.kernel,
    out_type=jax.ShapeDtypeStruct((NC, ND, H), jnp.float32),
    mesh=_sc_mesh(),
    scratch_types=[
        pltpu.VMEM((C, K), jnp.int32),
        pltpu.VMEM((C, K), jnp.int32),
        [pltpu.VMEM((K, H), jnp.float32) for _ in range(NB)],
        pltpu.VMEM_SHARED((AP, H), jnp.float32),
        [pltpu.SemaphoreType.DMA for _ in range(NB)],
    ],
)
def _sc_segsum(table, srcs, dstm, zeros, out, src_v, dst_v, rows, accum, sems):
    cid = lax.axis_index("c")
    sid = lax.axis_index("s")
    pltpu.sync_copy(zeros.at[pl.ds(sid * ZS, ZS)],
                    accum.at[pl.ds(sid * ZS, ZS)])
    pltpu.sync_copy(srcs.at[sid], src_v)
    pltpu.sync_copy(dstm.at[cid].at[sid], dst_v)
    plsc.subcore_barrier()
    for b in range(NB):
        pltpu.async_copy(table.at[src_v.at[b]], rows[b], sems[b])

    def outer(i, _):
        j0 = i * NB
        for b in range(NB):
            j = j0 + b
            pltpu.make_async_copy(table.at[src_v.at[b]], rows[b],
                                  sems[b]).wait()
            pltpu.sync_copy(rows[b], accum.at[dst_v.at[j]], add=True)

            @pl.when(j + NB < C)
            def _():
                pltpu.async_copy(table.at[src_v.at[j + NB]], rows[b], sems[b])
        return ()

    lax.fori_loop(0, C // NB, outer, (), unroll=False)
    plsc.subcore_barrier()
    pltpu.sync_copy(accum.at[pl.ds(sid * DS, DS)],
                    out.at[cid, pl.ds(sid * DS, DS)])


def _tc_remap_body(d_ref, o_ref):
    d = d_ref[...]
    o_ref[0] = jnp.where(d < ND, d, ND)
    o_ref[1] = jnp.where(d >= ND, d - ND, ND)


def _tc_remap(dst2d):
    return pl.pallas_call(
        _tc_remap_body,
        out_shape=jax.ShapeDtypeStruct((NC, EB, 128), jnp.int32),
    )(dst2d)


def _tc_sage1_body(p_ref, x_ref, degp_ref, wl_ref, bl_ref, wr_ref, o_ref):
    inv = 1.0 / jnp.maximum(degp_ref[:, 0:1], 1.0)
    mean = p_ref[...] * inv
    h = lax.dot_general(mean, wl_ref[...], (((1,), (1,)), ((), ())),
                        preferred_element_type=jnp.float32)
    h = h + bl_ref[...]
    h = h + lax.dot_general(x_ref[...], wr_ref[...], (((1,), (1,)), ((), ())),
                            preferred_element_type=jnp.float32)
    o_ref[...] = jnp.maximum(h, 0.0)


def _tc_sage1(p, xt, degp, Wl, bl, Wr):
    return pl.pallas_call(
        _tc_sage1_body,
        grid=(G,),
        in_specs=[
            pl.BlockSpec((R, H), lambda i: (i, 0)),
            pl.BlockSpec((R, H), lambda i: (i, 0)),
            pl.BlockSpec((R, H), lambda i: (i, 0)),
            pl.BlockSpec((H, H), lambda i: (0, 0)),
            pl.BlockSpec((1, H), lambda i: (0, 0)),
            pl.BlockSpec((H, H), lambda i: (0, 0)),
        ],
        out_specs=pl.BlockSpec((R, H), lambda i: (i, 0)),
        out_shape=jax.ShapeDtypeStruct((NPAD, H), jnp.float32),
    )(p, xt, degp, Wl, bl, Wr)


def _tc_step_body(p_ref, h1_ref, degp_ref, hprev_ref,
                  wl_ref, bl_ref, wr_ref, g_ref, b_ref,
                  wi_ref, bi_ref, wh_ref, bh_ref,
                  tw1_ref, tb1_ref, tw2_ref,
                  pw1_ref, pb1_ref, pw2_ref, b2_ref,
                  hout_ref, o8_ref):
    inv = 1.0 / jnp.maximum(degp_ref[:, 0:1], 1.0)
    mean = p_ref[...] * inv
    h = lax.dot_general(mean, wl_ref[...], (((1,), (1,)), ((), ())),
                        preferred_element_type=jnp.float32)
    h = h + bl_ref[...]
    h = h + lax.dot_general(h1_ref[...], wr_ref[...], (((1,), (1,)), ((), ())),
                            preferred_element_type=jnp.float32)
    h = jnp.maximum(h, 0.0)
    mu = jnp.mean(h, axis=-1, keepdims=True)
    var = jnp.mean((h - mu) ** 2, axis=-1, keepdims=True)
    e = (h - mu) / jnp.sqrt(var + 1e-5) * g_ref[...] + b_ref[...]
    hprev = hprev_ref[...]
    gi = lax.dot_general(e, wi_ref[...], (((1,), (1,)), ((), ())),
                         preferred_element_type=jnp.float32) + bi_ref[...]
    gh = lax.dot_general(hprev, wh_ref[...], (((1,), (1,)), ((), ())),
                         preferred_element_type=jnp.float32) + bh_ref[...]
    r = jax.nn.sigmoid(gi[:, 0:H] + gh[:, 0:H])
    z = jax.nn.sigmoid(gi[:, H:2 * H] + gh[:, H:2 * H])
    n = jnp.tanh(gi[:, 2 * H:] + r * gh[:, 2 * H:])
    hn = (1.0 - z) * n + z * hprev
    hout_ref[...] = hn
    zt = jnp.maximum(lax.dot_general(hn, tw1_ref[...], (((1,), (1,)), ((), ())),
                                     preferred_element_type=jnp.float32)
                     + tb1_ref[...], 0.0)
    thick = lax.dot_general(zt, tw2_ref[...], (((1,), (1,)), ((), ())),
                            preferred_element_type=jnp.float32)
    zp = jnp.maximum(lax.dot_general(hn, pw1_ref[...], (((1,), (1,)), ((), ())),
                                     preferred_element_type=jnp.float32)
                     + pb1_ref[...], 0.0)
    phys = lax.dot_general(zp, pw2_ref[...], (((1,), (1,)), ((), ())),
                           preferred_element_type=jnp.float32)
    o8_ref[...] = thick + phys + b2_ref[...]


def _tc_step(p, h1, degp, hprev, Wl2, bl2, Wr2, ln_g, ln_b,
             Wi, bi, Wh, bh, tW1, tb1, tW2p, pW1, pb1, pW2p, b2):
    full = lambda shape: pl.BlockSpec(shape, lambda i: tuple(0 for _ in shape))
    return pl.pallas_call(
        _tc_step_body,
        grid=(G,),
        in_specs=[
            pl.BlockSpec((R, H), lambda i: (i, 0)),
            pl.BlockSpec((R, H), lambda i: (i, 0)),
            pl.BlockSpec((R, H), lambda i: (i, 0)),
            pl.BlockSpec((R, H), lambda i: (i, 0)),
            full((H, H)), full((1, H)), full((H, H)),
            full((1, H)), full((1, H)),
            full((3 * H, H)), full((1, 3 * H)),
            full((3 * H, H)), full((1, 3 * H)),
            full((H, H)), full((1, H)), full((128, H)),
            full((H, H)), full((1, H)), full((128, H)), full((1, 128)),
        ],
        out_specs=[
            pl.BlockSpec((R, H), lambda i: (i, 0)),
            pl.BlockSpec((R, 128), lambda i: (i, 0)),
        ],
        out_shape=[
            jax.ShapeDtypeStruct((NPAD, H), jnp.float32),
            jax.ShapeDtypeStruct((NPAD, 128), jnp.float32),
        ],
    )(p, h1, degp, hprev, Wl2, bl2, Wr2, ln_g, ln_b,
      Wi, bi, Wh, bh, tW1, tb1, tW2p, pW1, pb1, pW2p, b2)


def kernel(x, edge_index, Wl1, bl1, Wr1, Wl2, bl2, Wr2, ln_g, ln_b,
           Wi, bi, Wh, bh, tW1, tb1, tW2, tb2, pW1, pb1, pW2, pb2):
    src = edge_index[0].reshape(NS, C, K)
    dstm = _tc_remap(edge_index[1].reshape(EB, 128)).reshape(NC, NS, C, K)
    zeros = jnp.zeros((AP, H), jnp.float32)
    ones_table = jnp.ones((NPAD, H), jnp.float32)
    degp = _sc_segsum(ones_table, src, dstm, zeros).reshape(NPAD, H)
    xp = jnp.pad(x, ((0, 0), (0, NPAD - N), (0, 0)))

    bl1r = bl1.reshape(1, H)
    bl2r = bl2.reshape(1, H)
    gr = ln_g.reshape(1, H)
    br = ln_b.reshape(1, H)
    bir = bi.reshape(1, 3 * H)
    bhr = bh.reshape(1, 3 * H)
    tb1r = tb1.reshape(1, H)
    pb1r = pb1.reshape(1, H)
    tW2p = jnp.zeros((128, H), jnp.float32).at[0:1].set(tW2)
    pW2p = jnp.zeros((128, H), jnp.float32).at[1:8].set(pW2)
    b2 = jnp.concatenate([tb2, pb2, jnp.zeros((120,), jnp.float32)]
                         ).reshape(1, 128)

    def half_step(carry, i):
        hprev, table = carry
        p = _sc_segsum(table, src, dstm, zeros).reshape(NPAD, H)
        t = i // 2

        def phase1(_):
            h1 = _tc_sage1(p, table, degp, Wl1, bl1r, Wr1)
            return hprev, h1, jnp.zeros((NPAD, 128), jnp.float32)

        def phase2(_):
            hn, o8 = _tc_step(p, table, degp, hprev, Wl2, bl2r, Wr2, gr, br,
                              Wi, bir, Wh, bhr, tW1, tb1r, tW2p,
                              pW1, pb1r, pW2p, b2)
            tnext = lax.dynamic_slice(
                xp, (jnp.minimum(t + 1, T - 1), 0, 0), (1, NPAD, H))[0]
            return hn, tnext, o8

        hnew, tnew, o8 = lax.cond(i % 2 == 0, phase1, phase2, None)
        return (hnew, tnew), o8

    h0 = jnp.zeros((NPAD, H), jnp.float32)
    (_, _), out8 = lax.scan(half_step, (h0, xp[0]), jnp.arange(2 * T))
    out8 = out8[1::2, :N]
    return out8[..., 0:1], out8[..., 1:8]


# speedup vs baseline: 11.5474x; 1.5429x over previous
import functools

import jax
import jax.numpy as jnp
from jax import lax
from jax.experimental import pallas as pl
from jax.experimental.pallas import tpu as pltpu
from jax.experimental.pallas import tpu_sc as plsc

T, N, E, F, H = 20, 10000, 320000, 128, 128
NC, NS = 2, 16
K = 80
CS = (E // NC // NS) // K
NPAD = 10240
ZR = NPAD // NS
R = 2048
G = NPAD // R


def _sc_mesh():
    return plsc.VectorSubcoreMesh(
        core_axis_name="c", subcore_axis_name="s", num_cores=NC, num_subcores=NS
    )


@functools.partial(
    pl.kernel,
    out_type=jax.ShapeDtypeStruct((NC, NPAD, H), jnp.float32),
    mesh=_sc_mesh(),
    scratch_types=[
        [pltpu.VMEM((2, K), jnp.int32) for _ in range(2)],
        [pltpu.VMEM((K, H), jnp.float32) for _ in range(2)],
        pltpu.VMEM_SHARED((NPAD, H), jnp.float32),
        [pltpu.SemaphoreType.DMA for _ in range(2)],
        [pltpu.SemaphoreType.DMA for _ in range(2)],
    ],
)
def _sc_segsum(table, idx, zeros, out, idx_v, rows, accum, isems, rsems):
    cid = lax.axis_index("c")
    sid = lax.axis_index("s")
    pltpu.sync_copy(zeros.at[pl.ds(sid * ZR, ZR)],
                    accum.at[pl.ds(sid * ZR, ZR)])
    plsc.subcore_barrier()
    my = idx.at[cid].at[sid]
    pltpu.async_copy(my.at[0], idx_v[0], isems[0])
    pltpu.async_copy(my.at[1], idx_v[1], isems[1])
    pltpu.make_async_copy(my.at[0], idx_v[0], isems[0]).wait()
    pltpu.async_copy(table.at[idx_v[0].at[0]], rows[0], rsems[0])

    def pair(i, _):
        for b in range(2):
            j = 2 * i + b
            nb = 1 - b

            @pl.when(j + 1 < CS)
            def _():
                pltpu.make_async_copy(my.at[j + 1], idx_v[nb],
                                      isems[nb]).wait()
                pltpu.async_copy(table.at[idx_v[nb].at[0]], rows[nb],
                                 rsems[nb])

            pltpu.make_async_copy(table.at[idx_v[b].at[0]], rows[b],
                                  rsems[b]).wait()
            pltpu.sync_copy(rows[b], accum.at[idx_v[b].at[1]], add=True)

            @pl.when(j + 2 < CS)
            def _():
                pltpu.async_copy(my.at[j + 2], idx_v[b], isems[b])
        return ()

    lax.fori_loop(0, CS // 2, pair, (), unroll=False)
    if CS % 2 == 1:
        pltpu.make_async_copy(table.at[idx_v[0].at[0]], rows[0],
                              rsems[0]).wait()
        pltpu.sync_copy(rows[0], accum.at[idx_v[0].at[1]], add=True)
    plsc.subcore_barrier()
    pltpu.sync_copy(accum.at[pl.ds(sid * ZR, ZR)],
                    out.at[cid, pl.ds(sid * ZR, ZR)])


def _tc_sage1_body(p_ref, x_ref, degp_ref, wl_ref, bl_ref, wr_ref, o_ref):
    deg = degp_ref[0, :, 0:1] + degp_ref[1, :, 0:1]
    inv = 1.0 / jnp.maximum(deg, 1.0)
    mean = (p_ref[0] + p_ref[1]) * inv
    h = lax.dot_general(mean, wl_ref[...], (((1,), (1,)), ((), ())),
                        preferred_element_type=jnp.float32)
    h = h + bl_ref[...]
    h = h + lax.dot_general(x_ref[...], wr_ref[...], (((1,), (1,)), ((), ())),
                            preferred_element_type=jnp.float32)
    o_ref[...] = jnp.maximum(h, 0.0)


def _tc_sage1(p, xt, degp, Wl, bl, Wr):
    return pl.pallas_call(
        _tc_sage1_body,
        grid=(G,),
        in_specs=[
            pl.BlockSpec((NC, R, H), lambda i: (0, i, 0)),
            pl.BlockSpec((R, H), lambda i: (i, 0)),
            pl.BlockSpec((NC, R, H), lambda i: (0, i, 0)),
            pl.BlockSpec((H, H), lambda i: (0, 0)),
            pl.BlockSpec((1, H), lambda i: (0, 0)),
            pl.BlockSpec((H, H), lambda i: (0, 0)),
        ],
        out_specs=pl.BlockSpec((R, H), lambda i: (i, 0)),
        out_shape=jax.ShapeDtypeStruct((NPAD, H), jnp.float32),
    )(p, xt, degp, Wl, bl, Wr)


def _tc_step_body(p_ref, h1_ref, degp_ref, hprev_ref,
                  wl_ref, bl_ref, wr_ref, g_ref, b_ref,
                  wi_ref, bi_ref, wh_ref, bh_ref,
                  tw1_ref, tb1_ref, tw2_ref,
                  pw1_ref, pb1_ref, pw2_ref, b2_ref,
                  hout_ref, o8_ref):
    deg = degp_ref[0, :, 0:1] + degp_ref[1, :, 0:1]
    inv = 1.0 / jnp.maximum(deg, 1.0)
    mean = (p_ref[0] + p_ref[1]) * inv
    h = lax.dot_general(mean, wl_ref[...], (((1,), (1,)), ((), ())),
                        preferred_element_type=jnp.float32)
    h = h + bl_ref[...]
    h = h + lax.dot_general(h1_ref[...], wr_ref[...], (((1,), (1,)), ((), ())),
                            preferred_element_type=jnp.float32)
    h = jnp.maximum(h, 0.0)
    mu = jnp.mean(h, axis=-1, keepdims=True)
    var = jnp.mean((h - mu) ** 2, axis=-1, keepdims=True)
    e = (h - mu) / jnp.sqrt(var + 1e-5) * g_ref[...] + b_ref[...]
    hprev = hprev_ref[...]
    gi = lax.dot_general(e, wi_ref[...], (((1,), (1,)), ((), ())),
                         preferred_element_type=jnp.float32) + bi_ref[...]
    gh = lax.dot_general(hprev, wh_ref[...], (((1,), (1,)), ((), ())),
                         preferred_element_type=jnp.float32) + bh_ref[...]
    r = jax.nn.sigmoid(gi[:, 0:H] + gh[:, 0:H])
    z = jax.nn.sigmoid(gi[:, H:2 * H] + gh[:, H:2 * H])
    n = jnp.tanh(gi[:, 2 * H:] + r * gh[:, 2 * H:])
    hn = (1.0 - z) * n + z * hprev
    hout_ref[...] = hn
    zt = jnp.maximum(lax.dot_general(hn, tw1_ref[...], (((1,), (1,)), ((), ())),
                                     preferred_element_type=jnp.float32)
                     + tb1_ref[...], 0.0)
    thick = lax.dot_general(zt, tw2_ref[...], (((1,), (1,)), ((), ())),
                            preferred_element_type=jnp.float32)
    zp = jnp.maximum(lax.dot_general(hn, pw1_ref[...], (((1,), (1,)), ((), ())),
                                     preferred_element_type=jnp.float32)
                     + pb1_ref[...], 0.0)
    phys = lax.dot_general(zp, pw2_ref[...], (((1,), (1,)), ((), ())),
                           preferred_element_type=jnp.float32)
    o8_ref[...] = thick + phys + b2_ref[...]


def _tc_step(p, h1, degp, hprev, Wl2, bl2, Wr2, ln_g, ln_b,
             Wi, bi, Wh, bh, tW1, tb1, tW2p, pW1, pb1, pW2p, b2):
    full = lambda shape: pl.BlockSpec(shape, lambda i: tuple(0 for _ in shape))
    return pl.pallas_call(
        _tc_step_body,
        grid=(G,),
        in_specs=[
            pl.BlockSpec((NC, R, H), lambda i: (0, i, 0)),
            pl.BlockSpec((R, H), lambda i: (i, 0)),
            pl.BlockSpec((NC, R, H), lambda i: (0, i, 0)),
            pl.BlockSpec((R, H), lambda i: (i, 0)),
            full((H, H)), full((1, H)), full((H, H)),
            full((1, H)), full((1, H)),
            full((3 * H, H)), full((1, 3 * H)),
            full((3 * H, H)), full((1, 3 * H)),
            full((H, H)), full((1, H)), full((128, H)),
            full((H, H)), full((1, H)), full((128, H)), full((1, 128)),
        ],
        out_specs=[
            pl.BlockSpec((R, H), lambda i: (i, 0)),
            pl.BlockSpec((R, 128), lambda i: (i, 0)),
        ],
        out_shape=[
            jax.ShapeDtypeStruct((NPAD, H), jnp.float32),
            jax.ShapeDtypeStruct((NPAD, 128), jnp.float32),
        ],
    )(p, h1, degp, hprev, Wl2, bl2, Wr2, ln_g, ln_b,
      Wi, bi, Wh, bh, tW1, tb1, tW2p, pW1, pb1, pW2p, b2)


def kernel(x, edge_index, Wl1, bl1, Wr1, Wl2, bl2, Wr2, ln_g, ln_b,
           Wi, bi, Wh, bh, tW1, tb1, tW2, tb2, pW1, pb1, pW2, pb2):
    idx = jnp.transpose(edge_index.reshape(2, NC, NS, CS, K), (1, 2, 3, 0, 4))
    zeros = jnp.zeros((NPAD, H), jnp.float32)
    ones_table = jnp.ones((NPAD, H), jnp.float32)
    degp = _sc_segsum(ones_table, idx, zeros)
    xp = jnp.pad(x, ((0, 0), (0, NPAD - N), (0, 0)))

    bl1r = bl1.reshape(1, H)
    bl2r = bl2.reshape(1, H)
    gr = ln_g.reshape(1, H)
    br = ln_b.reshape(1, H)
    bir = bi.reshape(1, 3 * H)
    bhr = bh.reshape(1, 3 * H)
    tb1r = tb1.reshape(1, H)
    pb1r = pb1.reshape(1, H)
    tW2p = jnp.zeros((128, H), jnp.float32).at[0:1].set(tW2)
    pW2p = jnp.zeros((128, H), jnp.float32).at[1:8].set(pW2)
    b2 = jnp.concatenate([tb2, pb2, jnp.zeros((120,), jnp.float32)]
                         ).reshape(1, 128)

    def half_step(carry, i):
        hprev, table = carry
        p = _sc_segsum(table, idx, zeros)
        t = i // 2

        def phase1(_):
            h1 = _tc_sage1(p, table, degp, Wl1, bl1r, Wr1)
            return hprev, h1, jnp.zeros((NPAD, 128), jnp.float32)

        def phase2(_):
            hn, o8 = _tc_step(p, table, degp, hprev, Wl2, bl2r, Wr2, gr, br,
                              Wi, bir, Wh, bhr, tW1, tb1r, tW2p,
                              pW1, pb1r, pW2p, b2)
            tnext = lax.dynamic_slice(
                xp, (jnp.minimum(t + 1, T - 1), 0, 0), (1, NPAD, H))[0]
            return hn, tnext, o8

        hnew, tnew, o8 = lax.cond(i % 2 == 0, phase1, phase2, None)
        return (hnew, tnew), o8

    h0 = jnp.zeros((NPAD, H), jnp.float32)
    (_, _), out8 = lax.scan(half_step, (h0, xp[0]), jnp.arange(2 * T))
    out8 = out8[1::2, :N]
    return out8[..., 0:1], out8[..., 1:8]


# speedup vs baseline: 14.1004x; 1.2211x over previous
import functools

import jax
import jax.numpy as jnp
from jax import lax
from jax.experimental import pallas as pl
from jax.experimental.pallas import tpu as pltpu
from jax.experimental.pallas import tpu_sc as plsc

T, N, E, F, H = 20, 10000, 320000, 128, 128
NC, NS = 2, 16
K = 100
CS = (E // NC // NS) // K
NPAD = 10240
ZR = NPAD // NS
R = 2048
G = NPAD // R


def _sc_mesh():
    return plsc.VectorSubcoreMesh(
        core_axis_name="c", subcore_axis_name="s", num_cores=NC, num_subcores=NS
    )


@functools.partial(
    pl.kernel,
    out_type=jax.ShapeDtypeStruct((NC, NPAD, H), jnp.float32),
    mesh=_sc_mesh(),
    scratch_types=[
        [pltpu.VMEM((2, K), jnp.int32) for _ in range(2)],
        [pltpu.VMEM((K, H), jnp.float32) for _ in range(2)],
        pltpu.VMEM_SHARED((NPAD, H), jnp.float32),
        [pltpu.SemaphoreType.DMA for _ in range(2)],
        [pltpu.SemaphoreType.DMA for _ in range(2)],
    ],
)
def _sc_segsum(table, idx, zeros, out, idx_v, rows, accum, isems, rsems):
    cid = lax.axis_index("c")
    sid = lax.axis_index("s")
    pltpu.sync_copy(zeros.at[pl.ds(sid * ZR, ZR)],
                    accum.at[pl.ds(sid * ZR, ZR)])
    plsc.subcore_barrier()
    my = idx.at[cid].at[sid]
    pltpu.async_copy(my.at[0], idx_v[0], isems[0])
    pltpu.async_copy(my.at[1], idx_v[1], isems[1])
    pltpu.make_async_copy(my.at[0], idx_v[0], isems[0]).wait()
    pltpu.async_copy(table.at[idx_v[0].at[0]], rows[0], rsems[0])

    def pair(i, _):
        for b in range(2):
            j = 2 * i + b
            nb = 1 - b

            @pl.when(j + 1 < CS)
            def _():
                pltpu.make_async_copy(my.at[j + 1], idx_v[nb],
                                      isems[nb]).wait()
                pltpu.async_copy(table.at[idx_v[nb].at[0]], rows[nb],
                                 rsems[nb])

            pltpu.make_async_copy(table.at[idx_v[b].at[0]], rows[b],
                                  rsems[b]).wait()
            pltpu.sync_copy(rows[b], accum.at[idx_v[b].at[1]], add=True)

            @pl.when(j + 2 < CS)
            def _():
                pltpu.async_copy(my.at[j + 2], idx_v[b], isems[b])
        return ()

    lax.fori_loop(0, CS // 2, pair, (), unroll=False)
    if CS % 2 == 1:
        pltpu.make_async_copy(table.at[idx_v[0].at[0]], rows[0],
                              rsems[0]).wait()
        pltpu.sync_copy(rows[0], accum.at[idx_v[0].at[1]], add=True)
    plsc.subcore_barrier()
    pltpu.sync_copy(accum.at[pl.ds(sid * ZR, ZR)],
                    out.at[cid, pl.ds(sid * ZR, ZR)])


def _tc_sage1_body(p_ref, x_ref, degp_ref, wl_ref, bl_ref, wr_ref, o_ref):
    deg = degp_ref[0, :, 0:1] + degp_ref[1, :, 0:1]
    inv = 1.0 / jnp.maximum(deg, 1.0)
    mean = (p_ref[0] + p_ref[1]) * inv
    h = lax.dot_general(mean, wl_ref[...], (((1,), (1,)), ((), ())),
                        preferred_element_type=jnp.float32)
    h = h + bl_ref[...]
    h = h + lax.dot_general(x_ref[...], wr_ref[...], (((1,), (1,)), ((), ())),
                            preferred_element_type=jnp.float32)
    o_ref[...] = jnp.maximum(h, 0.0)


def _tc_sage1(p, xt, degp, Wl, bl, Wr):
    return pl.pallas_call(
        _tc_sage1_body,
        grid=(G,),
        in_specs=[
            pl.BlockSpec((NC, R, H), lambda i: (0, i, 0)),
            pl.BlockSpec((R, H), lambda i: (i, 0)),
            pl.BlockSpec((NC, R, H), lambda i: (0, i, 0)),
            pl.BlockSpec((H, H), lambda i: (0, 0)),
            pl.BlockSpec((1, H), lambda i: (0, 0)),
            pl.BlockSpec((H, H), lambda i: (0, 0)),
        ],
        out_specs=pl.BlockSpec((R, H), lambda i: (i, 0)),
        out_shape=jax.ShapeDtypeStruct((NPAD, H), jnp.float32),
    )(p, xt, degp, Wl, bl, Wr)


def _tc_step_body(p_ref, h1_ref, degp_ref, hprev_ref,
                  wl_ref, bl_ref, wr_ref, g_ref, b_ref,
                  wi_ref, bi_ref, wh_ref, bh_ref,
                  tw1_ref, tb1_ref, tw2_ref,
                  pw1_ref, pb1_ref, pw2_ref, b2_ref,
                  hout_ref, o8_ref):
    deg = degp_ref[0, :, 0:1] + degp_ref[1, :, 0:1]
    inv = 1.0 / jnp.maximum(deg, 1.0)
    mean = (p_ref[0] + p_ref[1]) * inv
    h = lax.dot_general(mean, wl_ref[...], (((1,), (1,)), ((), ())),
                        preferred_element_type=jnp.float32)
    h = h + bl_ref[...]
    h = h + lax.dot_general(h1_ref[...], wr_ref[...], (((1,), (1,)), ((), ())),
                            preferred_element_type=jnp.float32)
    h = jnp.maximum(h, 0.0)
    mu = jnp.mean(h, axis=-1, keepdims=True)
    var = jnp.mean((h - mu) ** 2, axis=-1, keepdims=True)
    e = (h - mu) / jnp.sqrt(var + 1e-5) * g_ref[...] + b_ref[...]
    hprev = hprev_ref[...]
    gi = lax.dot_general(e, wi_ref[...], (((1,), (1,)), ((), ())),
                         preferred_element_type=jnp.float32) + bi_ref[...]
    gh = lax.dot_general(hprev, wh_ref[...], (((1,), (1,)), ((), ())),
                         preferred_element_type=jnp.float32) + bh_ref[...]
    r = jax.nn.sigmoid(gi[:, 0:H] + gh[:, 0:H])
    z = jax.nn.sigmoid(gi[:, H:2 * H] + gh[:, H:2 * H])
    n = jnp.tanh(gi[:, 2 * H:] + r * gh[:, 2 * H:])
    hn = (1.0 - z) * n + z * hprev
    hout_ref[...] = hn
    zt = jnp.maximum(lax.dot_general(hn, tw1_ref[...], (((1,), (1,)), ((), ())),
                                     preferred_element_type=jnp.float32)
                     + tb1_ref[...], 0.0)
    thick = lax.dot_general(zt, tw2_ref[...], (((1,), (1,)), ((), ())),
                            preferred_element_type=jnp.float32)
    zp = jnp.maximum(lax.dot_general(hn, pw1_ref[...], (((1,), (1,)), ((), ())),
                                     preferred_element_type=jnp.float32)
                     + pb1_ref[...], 0.0)
    phys = lax.dot_general(zp, pw2_ref[...], (((1,), (1,)), ((), ())),
                           preferred_element_type=jnp.float32)
    o8_ref[...] = thick + phys + b2_ref[...]


def _tc_step(p, h1, degp, hprev, Wl2, bl2, Wr2, ln_g, ln_b,
             Wi, bi, Wh, bh, tW1, tb1, tW2p, pW1, pb1, pW2p, b2):
    full = lambda shape: pl.BlockSpec(shape, lambda i: tuple(0 for _ in shape))
    return pl.pallas_call(
        _tc_step_body,
        grid=(G,),
        in_specs=[
            pl.BlockSpec((NC, R, H), lambda i: (0, i, 0)),
            pl.BlockSpec((R, H), lambda i: (i, 0)),
            pl.BlockSpec((NC, R, H), lambda i: (0, i, 0)),
            pl.BlockSpec((R, H), lambda i: (i, 0)),
            full((H, H)), full((1, H)), full((H, H)),
            full((1, H)), full((1, H)),
            full((3 * H, H)), full((1, 3 * H)),
            full((3 * H, H)), full((1, 3 * H)),
            full((H, H)), full((1, H)), full((128, H)),
            full((H, H)), full((1, H)), full((128, H)), full((1, 128)),
        ],
        out_specs=[
            pl.BlockSpec((R, H), lambda i: (i, 0)),
            pl.BlockSpec((R, 128), lambda i: (i, 0)),
        ],
        out_shape=[
            jax.ShapeDtypeStruct((NPAD, H), jnp.float32),
            jax.ShapeDtypeStruct((NPAD, 128), jnp.float32),
        ],
    )(p, h1, degp, hprev, Wl2, bl2, Wr2, ln_g, ln_b,
      Wi, bi, Wh, bh, tW1, tb1, tW2p, pW1, pb1, pW2p, b2)


def kernel(x, edge_index, Wl1, bl1, Wr1, Wl2, bl2, Wr2, ln_g, ln_b,
           Wi, bi, Wh, bh, tW1, tb1, tW2, tb2, pW1, pb1, pW2, pb2):
    idx = jnp.transpose(edge_index.reshape(2, NC, NS, CS, K), (1, 2, 3, 0, 4))
    zeros = jnp.zeros((NPAD, H), jnp.float32)
    ones_table = jnp.ones((NPAD, H), jnp.float32)
    degp = _sc_segsum(ones_table, idx, zeros)
    xp = jnp.pad(x, ((0, 0), (0, NPAD - N), (0, 0)))

    bl1r = bl1.reshape(1, H)
    bl2r = bl2.reshape(1, H)
    gr = ln_g.reshape(1, H)
    br = ln_b.reshape(1, H)
    bir = bi.reshape(1, 3 * H)
    bhr = bh.reshape(1, 3 * H)
    tb1r = tb1.reshape(1, H)
    pb1r = pb1.reshape(1, H)
    tW2p = jnp.zeros((128, H), jnp.float32).at[0:1].set(tW2)
    pW2p = jnp.zeros((128, H), jnp.float32).at[1:8].set(pW2)
    b2 = jnp.concatenate([tb2, pb2, jnp.zeros((120,), jnp.float32)]
                         ).reshape(1, 128)

    def step_t(carry, t):
        hprev, p1 = carry
        xt = lax.dynamic_slice(xp, (t, 0, 0), (1, NPAD, H))[0]
        h1 = _tc_sage1(p1, xt, degp, Wl1, bl1r, Wr1)
        p2 = _sc_segsum(h1, idx, zeros)
        hn, o8 = _tc_step(p2, h1, degp, hprev, Wl2, bl2r, Wr2, gr, br,
                          Wi, bir, Wh, bhr, tW1, tb1r, tW2p,
                          pW1, pb1r, pW2p, b2)
        xnext = lax.dynamic_slice(
            xp, (jnp.minimum(t + 1, T - 1), 0, 0), (1, NPAD, H))[0]
        p1n = _sc_segsum(xnext, idx, zeros)
        return (hn, p1n), o8

    h0 = jnp.zeros((NPAD, H), jnp.float32)
    p1_0 = _sc_segsum(xp[0], idx, zeros)
    (_, _), out8 = lax.scan(step_t, (h0, p1_0), jnp.arange(T))
    out8 = out8[:, :N]
    return out8[..., 0:1], out8[..., 1:8]


# speedup vs baseline: 15.2028x; 1.0782x over previous
import functools

import jax
import jax.numpy as jnp
from jax import lax
from jax.experimental import pallas as pl
from jax.experimental.pallas import tpu as pltpu
from jax.experimental.pallas import tpu_sc as plsc

T, N, E, F, H = 20, 10000, 320000, 128, 128
NC, NS = 2, 16
K = 125
CS = (E // NC // NS) // K
NPAD = 10240
ZR = NPAD // NS
R = 2048
G = NPAD // R


def _sc_mesh():
    return plsc.VectorSubcoreMesh(
        core_axis_name="c", subcore_axis_name="s", num_cores=NC, num_subcores=NS
    )


@functools.partial(
    pl.kernel,
    out_type=jax.ShapeDtypeStruct((NC, NPAD, H), jnp.float32),
    mesh=_sc_mesh(),
    scratch_types=[
        [pltpu.VMEM((2, K), jnp.int32) for _ in range(2)],
        [pltpu.VMEM((K, H), jnp.float32) for _ in range(2)],
        pltpu.VMEM_SHARED((NPAD, H), jnp.float32),
        [pltpu.SemaphoreType.DMA for _ in range(2)],
        [pltpu.SemaphoreType.DMA for _ in range(2)],
    ],
)
def _sc_segsum(table, idx, zeros, out, idx_v, rows, accum, isems, rsems):
    cid = lax.axis_index("c")
    sid = lax.axis_index("s")
    pltpu.sync_copy(zeros.at[pl.ds(sid * ZR, ZR)],
                    accum.at[pl.ds(sid * ZR, ZR)])
    plsc.subcore_barrier()
    my = idx.at[cid].at[sid]
    pltpu.async_copy(my.at[0], idx_v[0], isems[0])
    pltpu.async_copy(my.at[1], idx_v[1], isems[1])
    pltpu.make_async_copy(my.at[0], idx_v[0], isems[0]).wait()
    pltpu.async_copy(table.at[idx_v[0].at[0]], rows[0], rsems[0])

    def pair(i, _):
        for b in range(2):
            j = 2 * i + b
            nb = 1 - b

            @pl.when(j + 1 < CS)
            def _():
                pltpu.make_async_copy(my.at[j + 1], idx_v[nb],
                                      isems[nb]).wait()
                pltpu.async_copy(table.at[idx_v[nb].at[0]], rows[nb],
                                 rsems[nb])

            pltpu.make_async_copy(table.at[idx_v[b].at[0]], rows[b],
                                  rsems[b]).wait()
            pltpu.sync_copy(rows[b], accum.at[idx_v[b].at[1]], add=True)

            @pl.when(j + 2 < CS)
            def _():
                pltpu.async_copy(my.at[j + 2], idx_v[b], isems[b])
        return ()

    lax.fori_loop(0, CS // 2, pair, (), unroll=False)
    if CS % 2 == 1:
        pltpu.make_async_copy(table.at[idx_v[0].at[0]], rows[0],
                              rsems[0]).wait()
        pltpu.sync_copy(rows[0], accum.at[idx_v[0].at[1]], add=True)
    plsc.subcore_barrier()
    pltpu.sync_copy(accum.at[pl.ds(sid * ZR, ZR)],
                    out.at[cid, pl.ds(sid * ZR, ZR)])


def _tc_sage1_body(p_ref, x_ref, degp_ref, wl_ref, bl_ref, wr_ref, o_ref):
    deg = degp_ref[0, :, 0:1] + degp_ref[1, :, 0:1]
    inv = 1.0 / jnp.maximum(deg, 1.0)
    mean = (p_ref[0] + p_ref[1]) * inv
    h = lax.dot_general(mean, wl_ref[...], (((1,), (1,)), ((), ())),
                        preferred_element_type=jnp.float32)
    h = h + bl_ref[...]
    h = h + lax.dot_general(x_ref[...], wr_ref[...], (((1,), (1,)), ((), ())),
                            preferred_element_type=jnp.float32)
    o_ref[...] = jnp.maximum(h, 0.0)


def _tc_sage1(p, xt, degp, Wl, bl, Wr):
    return pl.pallas_call(
        _tc_sage1_body,
        grid=(G,),
        in_specs=[
            pl.BlockSpec((NC, R, H), lambda i: (0, i, 0)),
            pl.BlockSpec((R, H), lambda i: (i, 0)),
            pl.BlockSpec((NC, R, H), lambda i: (0, i, 0)),
            pl.BlockSpec((H, H), lambda i: (0, 0)),
            pl.BlockSpec((1, H), lambda i: (0, 0)),
            pl.BlockSpec((H, H), lambda i: (0, 0)),
        ],
        out_specs=pl.BlockSpec((R, H), lambda i: (i, 0)),
        out_shape=jax.ShapeDtypeStruct((NPAD, H), jnp.float32),
    )(p, xt, degp, Wl, bl, Wr)


def _tc_step_body(p_ref, h1_ref, degp_ref, hprev_ref,
                  wl_ref, bl_ref, wr_ref, g_ref, b_ref,
                  wi_ref, bi_ref, wh_ref, bh_ref,
                  tw1_ref, tb1_ref, tw2_ref,
                  pw1_ref, pb1_ref, pw2_ref, b2_ref,
                  hout_ref, o8_ref):
    deg = degp_ref[0, :, 0:1] + degp_ref[1, :, 0:1]
    inv = 1.0 / jnp.maximum(deg, 1.0)
    mean = (p_ref[0] + p_ref[1]) * inv
    h = lax.dot_general(mean, wl_ref[...], (((1,), (1,)), ((), ())),
                        preferred_element_type=jnp.float32)
    h = h + bl_ref[...]
    h = h + lax.dot_general(h1_ref[...], wr_ref[...], (((1,), (1,)), ((), ())),
                            preferred_element_type=jnp.float32)
    h = jnp.maximum(h, 0.0)
    mu = jnp.mean(h, axis=-1, keepdims=True)
    var = jnp.mean((h - mu) ** 2, axis=-1, keepdims=True)
    e = (h - mu) / jnp.sqrt(var + 1e-5) * g_ref[...] + b_ref[...]
    hprev = hprev_ref[...]
    gi = lax.dot_general(e, wi_ref[...], (((1,), (1,)), ((), ())),
                         preferred_element_type=jnp.float32) + bi_ref[...]
    gh = lax.dot_general(hprev, wh_ref[...], (((1,), (1,)), ((), ())),
                         preferred_element_type=jnp.float32) + bh_ref[...]
    r = jax.nn.sigmoid(gi[:, 0:H] + gh[:, 0:H])
    z = jax.nn.sigmoid(gi[:, H:2 * H] + gh[:, H:2 * H])
    n = jnp.tanh(gi[:, 2 * H:] + r * gh[:, 2 * H:])
    hn = (1.0 - z) * n + z * hprev
    hout_ref[...] = hn
    zt = jnp.maximum(lax.dot_general(hn, tw1_ref[...], (((1,), (1,)), ((), ())),
                                     preferred_element_type=jnp.float32)
                     + tb1_ref[...], 0.0)
    thick = lax.dot_general(zt, tw2_ref[...], (((1,), (1,)), ((), ())),
                            preferred_element_type=jnp.float32)
    zp = jnp.maximum(lax.dot_general(hn, pw1_ref[...], (((1,), (1,)), ((), ())),
                                     preferred_element_type=jnp.float32)
                     + pb1_ref[...], 0.0)
    phys = lax.dot_general(zp, pw2_ref[...], (((1,), (1,)), ((), ())),
                           preferred_element_type=jnp.float32)
    o8_ref[...] = thick + phys + b2_ref[...]


def _tc_step(p, h1, degp, hprev, Wl2, bl2, Wr2, ln_g, ln_b,
             Wi, bi, Wh, bh, tW1, tb1, tW2p, pW1, pb1, pW2p, b2):
    full = lambda shape: pl.BlockSpec(shape, lambda i: tuple(0 for _ in shape))
    return pl.pallas_call(
        _tc_step_body,
        grid=(G,),
        in_specs=[
            pl.BlockSpec((NC, R, H), lambda i: (0, i, 0)),
            pl.BlockSpec((R, H), lambda i: (i, 0)),
            pl.BlockSpec((NC, R, H), lambda i: (0, i, 0)),
            pl.BlockSpec((R, H), lambda i: (i, 0)),
            full((H, H)), full((1, H)), full((H, H)),
            full((1, H)), full((1, H)),
            full((3 * H, H)), full((1, 3 * H)),
            full((3 * H, H)), full((1, 3 * H)),
            full((H, H)), full((1, H)), full((128, H)),
            full((H, H)), full((1, H)), full((128, H)), full((1, 128)),
        ],
        out_specs=[
            pl.BlockSpec((R, H), lambda i: (i, 0)),
            pl.BlockSpec((R, 128), lambda i: (i, 0)),
        ],
        out_shape=[
            jax.ShapeDtypeStruct((NPAD, H), jnp.float32),
            jax.ShapeDtypeStruct((NPAD, 128), jnp.float32),
        ],
    )(p, h1, degp, hprev, Wl2, bl2, Wr2, ln_g, ln_b,
      Wi, bi, Wh, bh, tW1, tb1, tW2p, pW1, pb1, pW2p, b2)


def kernel(x, edge_index, Wl1, bl1, Wr1, Wl2, bl2, Wr2, ln_g, ln_b,
           Wi, bi, Wh, bh, tW1, tb1, tW2, tb2, pW1, pb1, pW2, pb2):
    idx = jnp.transpose(edge_index.reshape(2, NC, NS, CS, K), (1, 2, 3, 0, 4))
    zeros = jnp.zeros((NPAD, H), jnp.float32)
    ones_table = jnp.ones((NPAD, H), jnp.float32)
    degp = _sc_segsum(ones_table, idx, zeros)
    xp = jnp.pad(x, ((0, 0), (0, NPAD - N), (0, 0)))

    bl1r = bl1.reshape(1, H)
    bl2r = bl2.reshape(1, H)
    gr = ln_g.reshape(1, H)
    br = ln_b.reshape(1, H)
    bir = bi.reshape(1, 3 * H)
    bhr = bh.reshape(1, 3 * H)
    tb1r = tb1.reshape(1, H)
    pb1r = pb1.reshape(1, H)
    tW2p = jnp.zeros((128, H), jnp.float32).at[0:1].set(tW2)
    pW2p = jnp.zeros((128, H), jnp.float32).at[1:8].set(pW2)
    b2 = jnp.concatenate([tb2, pb2, jnp.zeros((120,), jnp.float32)]
                         ).reshape(1, 128)

    def step_t(carry, t):
        hprev, p1 = carry
        xt = lax.dynamic_slice(xp, (t, 0, 0), (1, NPAD, H))[0]
        h1 = _tc_sage1(p1, xt, degp, Wl1, bl1r, Wr1)
        p2 = _sc_segsum(h1, idx, zeros)
        hn, o8 = _tc_step(p2, h1, degp, hprev, Wl2, bl2r, Wr2, gr, br,
                          Wi, bir, Wh, bhr, tW1, tb1r, tW2p,
                          pW1, pb1r, pW2p, b2)
        xnext = lax.dynamic_slice(
            xp, (jnp.minimum(t + 1, T - 1), 0, 0), (1, NPAD, H))[0]
        p1n = _sc_segsum(xnext, idx, zeros)
        return (hn, p1n), o8

    h0 = jnp.zeros((NPAD, H), jnp.float32)
    p1_0 = _sc_segsum(xp[0], idx, zeros)
    (_, _), out8 = lax.scan(step_t, (h0, p1_0), jnp.arange(T))
    out8 = out8[:, :N]
    return out8[..., 0:1], out8[..., 1:8]
